# baseline (device time: 54516 ns/iter reference)
import jax
import jax.numpy as jnp
from jax import lax
from jax.experimental import pallas as pl
from jax.experimental.pallas import tpu as pltpu

N_DEV = 8


def kernel(x, Wg, Wu, Wd):
    m, k = x.shape
    _, h_per = Wg.shape
    chunk = m // N_DEV

    def body(x_ref, wg_ref, wu_ref, wd_ref, out_ref,
             p_ref, sbuf, comm, agsrc, agbuf,
             rs_send, rs_recv, ag_send, ag_recv):
        my = lax.axis_index("i")

        barrier = pltpu.get_barrier_semaphore()
        for p in range(1, N_DEV):
            pl.semaphore_signal(
                barrier, inc=1,
                device_id=(lax.rem(my + p, N_DEV),),
                device_id_type=pl.DeviceIdType.MESH,
            )
        pl.semaphore_wait(barrier, N_DEV - 1)

        def rs_desc(d):
            return pltpu.make_async_remote_copy(
                src_ref=sbuf.at[d], dst_ref=comm.at[d],
                send_sem=rs_send.at[d], recv_sem=rs_recv.at[d],
                device_id=(lax.rem(my + d, N_DEV),),
                device_id_type=pl.DeviceIdType.MESH,
            )

        def ag_desc(d):
            return pltpu.make_async_remote_copy(
                src_ref=agsrc, dst_ref=agbuf.at[d],
                send_sem=ag_send.at[d], recv_sem=ag_recv.at[d],
                device_id=(lax.rem(my + d, N_DEV),),
                device_id_type=pl.DeviceIdType.MESH,
            )

        nblk = 4
        bsz = 2 * chunk

        def block_partial(b):
            xb_b = x_ref[pl.ds(b * bsz, bsz), :]
            gate = jnp.dot(xb_b, wg_ref[...],
                           preferred_element_type=jnp.float32)
            up = jnp.dot(xb_b, wu_ref[...],
                         preferred_element_type=jnp.float32)
            hh = (gate * (up * jax.nn.sigmoid(up))).astype(jnp.bfloat16)
            return jnp.dot(hh, wd_ref[...],
                           preferred_element_type=jnp.float32)

        own_b = my // 2

        def send_block(bi, carry):
            b = lax.rem(own_b + bi, nblk)
            p_ref[...] = block_partial(b)
            for half in (0, 1):
                c = 2 * b + half
                d = lax.rem(c - my + N_DEV, N_DEV)
                sbuf[d, :, :] = p_ref[
                    half * chunk:(half + 1) * chunk, :
                ].astype(jnp.bfloat16)
                rs_desc(d).start()
            return carry

        lax.fori_loop(1, nblk, send_block, 0)

        p_ref[...] = block_partial(own_b)
        sib_half = 1 - lax.rem(my, 2)
        d_sib = lax.rem(2 * own_b + sib_half - my + N_DEV, N_DEV)
        sbuf[d_sib, :, :] = p_ref[
            pl.ds(sib_half * chunk, chunk), :
        ].astype(jnp.bfloat16)
        rs_desc(d_sib).start()
        own = p_ref[pl.ds(lax.rem(my, 2) * chunk, chunk), :]

        def red_step(d, red):
            rs_desc(d).wait_recv()
            return red + comm[d, :, :].astype(jnp.float32)

        red = lax.fori_loop(1, N_DEV, red_step, own)
        out_ref[pl.ds(my * chunk, chunk), :] = red

        agsrc[...] = red.astype(jnp.bfloat16)

        def ag_start(d, carry):
            ag_desc(d).start()
            return carry

        lax.fori_loop(1, N_DEV, ag_start, 0)

        def ag_wait(d, carry):
            ag_desc(d).wait_recv()
            c = lax.rem(my - d + N_DEV, N_DEV)
            out_ref[pl.ds(c * chunk, chunk), :] = agbuf[d, :, :].astype(
                jnp.float32
            )
            return carry

        lax.fori_loop(1, N_DEV, ag_wait, 0)

        def drain(d, carry):
            rs_desc(d).wait_send()
            ag_desc(d).wait_send()
            return carry

        lax.fori_loop(1, N_DEV, drain, 0)

    call = pl.pallas_call(
        body,
        out_shape=jax.ShapeDtypeStruct((m, k), jnp.float32),
        in_specs=[pl.BlockSpec(memory_space=pltpu.VMEM)] * 4,
        out_specs=pl.BlockSpec(memory_space=pltpu.VMEM),
        scratch_shapes=[
            pltpu.VMEM((2 * chunk, k), jnp.float32),
            pltpu.VMEM((N_DEV, chunk, k), jnp.bfloat16),
            pltpu.VMEM((N_DEV, chunk, k), jnp.bfloat16),
            pltpu.VMEM((chunk, k), jnp.bfloat16),
            pltpu.VMEM((N_DEV, chunk, k), jnp.bfloat16),
            pltpu.SemaphoreType.DMA((N_DEV,)),
            pltpu.SemaphoreType.DMA((N_DEV,)),
            pltpu.SemaphoreType.DMA((N_DEV,)),
            pltpu.SemaphoreType.DMA((N_DEV,)),
        ],
        compiler_params=pltpu.CompilerParams(collective_id=0),
    )
    return call(
        x.astype(jnp.bfloat16),
        Wg.astype(jnp.bfloat16),
        Wu.astype(jnp.bfloat16),
        Wd.astype(jnp.bfloat16),
    )


# device time: 54225 ns/iter; 1.0054x vs baseline; 1.0054x over previous
import jax
import jax.numpy as jnp
from jax import lax
from jax.experimental import pallas as pl
from jax.experimental.pallas import tpu as pltpu

N_DEV = 8

DO_RS = True
DO_AG = True


def kernel(x, Wg, Wu, Wd):
    m, k = x.shape
    _, h_per = Wg.shape
    chunk = m // N_DEV

    def body(x_ref, wg_ref, wu_ref, wd_ref, out_ref,
             sbuf, comm, agsrc, agbuf,
             rs_send, rs_recv, ag_send, ag_recv):
        my = lax.axis_index("i")

        if DO_RS or DO_AG:
            barrier = pltpu.get_barrier_semaphore()
            for p in range(1, N_DEV):
                pl.semaphore_signal(
                    barrier, inc=1,
                    device_id=(lax.rem(my + p, N_DEV),),
                    device_id_type=pl.DeviceIdType.MESH,
                )
            pl.semaphore_wait(barrier, N_DEV - 1)

        def rs_desc(d):
            return pltpu.make_async_remote_copy(
                src_ref=sbuf.at[d], dst_ref=comm.at[d],
                send_sem=rs_send.at[d], recv_sem=rs_recv.at[d],
                device_id=(lax.rem(my + d, N_DEV),),
                device_id_type=pl.DeviceIdType.MESH,
            )

        def ag_desc(d):
            return pltpu.make_async_remote_copy(
                src_ref=agsrc, dst_ref=agbuf.at[d],
                send_sem=ag_send.at[d], recv_sem=ag_recv.at[d],
                device_id=(lax.rem(my + d, N_DEV),),
                device_id_type=pl.DeviceIdType.MESH,
            )

        def mlp_chunk(c):
            xb_c = x_ref[pl.ds(c * chunk, chunk), :]
            gate = jnp.dot(xb_c, wg_ref[...],
                           preferred_element_type=jnp.float32)
            up = jnp.dot(xb_c, wu_ref[...],
                         preferred_element_type=jnp.float32)
            hh = (gate * (up * jax.nn.sigmoid(up))).astype(jnp.bfloat16)
            return jnp.dot(hh, wd_ref[...],
                           preferred_element_type=jnp.float32)

        def send_step(d, carry):
            sbuf[d, :, :] = mlp_chunk(lax.rem(my + d, N_DEV)).astype(
                jnp.bfloat16
            )
            if DO_RS:
                rs_desc(d).start()
            return carry

        lax.fori_loop(1, N_DEV, send_step, 0)
        own = mlp_chunk(my)

        def red_step(d, red):
            if DO_RS:
                rs_desc(d).wait_recv()
            return red + comm[d, :, :].astype(jnp.float32)

        red = lax.fori_loop(1, N_DEV, red_step, own)
        out_ref[pl.ds(my * chunk, chunk), :] = red

        agsrc[...] = red.astype(jnp.bfloat16)

        if DO_AG:
            def ag_start(d, carry):
                ag_desc(d).start()
                return carry

            lax.fori_loop(1, N_DEV, ag_start, 0)

            def ag_wait(d, carry):
                ag_desc(d).wait_recv()
                c = lax.rem(my - d + N_DEV, N_DEV)
                out_ref[pl.ds(c * chunk, chunk), :] = agbuf[d, :, :].astype(
                    jnp.float32
                )
                return carry

            lax.fori_loop(1, N_DEV, ag_wait, 0)

        def drain(d, carry):
            if DO_RS:
                rs_desc(d).wait_send()
            if DO_AG:
                ag_desc(d).wait_send()
            return carry

        if DO_RS or DO_AG:
            lax.fori_loop(1, N_DEV, drain, 0)

    call = pl.pallas_call(
        body,
        out_shape=jax.ShapeDtypeStruct((m, k), jnp.float32),
        in_specs=[pl.BlockSpec(memory_space=pltpu.VMEM)] * 4,
        out_specs=pl.BlockSpec(memory_space=pltpu.VMEM),
        scratch_shapes=[
            pltpu.VMEM((N_DEV, chunk, k), jnp.bfloat16),
            pltpu.VMEM((N_DEV, chunk, k), jnp.bfloat16),
            pltpu.VMEM((chunk, k), jnp.bfloat16),
            pltpu.VMEM((N_DEV, chunk, k), jnp.bfloat16),
            pltpu.SemaphoreType.DMA((N_DEV,)),
            pltpu.SemaphoreType.DMA((N_DEV,)),
            pltpu.SemaphoreType.DMA((N_DEV,)),
            pltpu.SemaphoreType.DMA((N_DEV,)),
        ],
        compiler_params=(
            pltpu.CompilerParams(collective_id=0)
            if (DO_RS or DO_AG) else pltpu.CompilerParams()
        ),
    )
    return call(
        x.astype(jnp.bfloat16),
        Wg.astype(jnp.bfloat16),
        Wu.astype(jnp.bfloat16),
        Wd.astype(jnp.bfloat16),
    )
